# Initial kernel scaffold; baseline (speedup 1.0000x reference)
import jax
import jax.numpy as jnp
from jax.experimental import pallas as pl


def kernel(x, edge_index, batch, W_rel1, b_rel1, W_root1, p1, W_rel2, b_rel2, W_root2, p2, W_l1, b_l1, W_l2, b_l2):
    raise NotImplementedError("write your pallas kernel here")


# speedup vs baseline: 9.0917x; 9.0917x over previous
import functools

import jax
import jax.numpy as jnp
from jax import lax
from jax.experimental import pallas as pl
from jax.experimental.pallas import tpu as pltpu
from jax.experimental.pallas import tpu_sc as plsc

N = 10000
E = 320000
F = 128
G = 128
C = 10

NPAD = 10752
PADG = 255
EPAD = 327680
NW = 32
EPW = EPAD // NW
ECH = 128
NCH = EPW // ECH
ROWS = NPAD // 16

MB = 512
RB = 256
JB = 256
PCH = 512


def _sc_scatter_add(x_pad, src2d, dst2d, zero_rows, interpret=False):
  mesh = plsc.VectorSubcoreMesh(core_axis_name="c", subcore_axis_name="s",
                                num_cores=2, num_subcores=16)

  @functools.partial(
      pl.kernel,
      out_type=(jax.ShapeDtypeStruct((NPAD, F), jnp.float32),
                jax.ShapeDtypeStruct((NPAD, F), jnp.float32)),
      mesh=mesh,
      scratch_types=[
          pltpu.VMEM((NCH, ECH), jnp.int32),
          pltpu.VMEM((NCH, ECH), jnp.int32),
          pltpu.VMEM((ECH, F), jnp.float32),
          pltpu.VMEM_SHARED((NPAD, F), jnp.float32),
          pltpu.SemaphoreType.DMA,
      ],
      interpret=interpret,
  )
  def k(x_hbm, src_hbm, dst_hbm, z_hbm, out_a, out_b, src_v, dst_v, rows_v,
        acc_sh, sem):
    c = lax.axis_index("c")
    s = lax.axis_index("s")
    wid = s * 2 + c
    pltpu.sync_copy(z_hbm, acc_sh.at[pl.ds(s * ROWS, ROWS)])
    base = wid * (EPW // ECH)
    pltpu.sync_copy(src_hbm.at[pl.ds(base, NCH)], src_v)
    pltpu.sync_copy(dst_hbm.at[pl.ds(base, NCH)], dst_v)
    plsc.subcore_barrier()

    def body(i, carry):
      pltpu.async_copy(x_hbm.at[src_v.at[i]], rows_v, sem).wait()
      pltpu.sync_copy(rows_v, acc_sh.at[dst_v.at[i]], add=True)
      return carry

    lax.fori_loop(0, NCH, body, 0)
    plsc.subcore_barrier()

    @pl.when(c == 0)
    def _():
      pltpu.sync_copy(acc_sh.at[pl.ds(s * ROWS, ROWS)],
                      out_a.at[pl.ds(s * ROWS, ROWS)])

    @pl.when(c == 1)
    def _():
      pltpu.sync_copy(acc_sh.at[pl.ds(s * ROWS, ROWS)],
                      out_b.at[pl.ds(s * ROWS, ROWS)])

  return k(x_pad, src2d, dst2d, zero_rows)


def _stats_body(batch_ref, n0_ref, start_ref, k1i_ref, k2i_ref, k1f_ref,
                k2f_ref):
  g = lax.broadcasted_iota(jnp.int32, (1, 256), 1)

  def body(i, carry):
    n0, st = carry
    blk = batch_ref[pl.ds(i * 256, 256), :]
    n0 = n0 + jnp.sum((blk == g).astype(jnp.int32), axis=0, keepdims=True)
    st = st + jnp.sum((blk < g).astype(jnp.int32), axis=0, keepdims=True)
    return n0, st

  z = jnp.zeros((1, 256), jnp.int32)
  n0, st = lax.fori_loop(0, NPAD // 256, body, (z, z))
  d1 = (4 * n0 + 4).astype(jnp.float32)
  k1f = jnp.floor(d1 * 0.2 + 0.001)
  k1i = k1f.astype(jnp.int32)
  d2 = (4 * k1i + 4).astype(jnp.float32)
  k2f = jnp.floor(d2 * 0.2 + 0.001)
  k2i = k2f.astype(jnp.int32)
  n0_ref[...] = n0
  start_ref[...] = st
  k1i_ref[...] = k1i
  k2i_ref[...] = k2i
  k1f_ref[...] = k1f
  k2f_ref[...] = k2f


def _stats(batch_col, interpret=False):
  o = jax.ShapeDtypeStruct((1, 256), jnp.int32)
  of = jax.ShapeDtypeStruct((1, 256), jnp.float32)
  return pl.pallas_call(
      _stats_body,
      out_shape=(o, o, o, o, of, of),
      interpret=interpret,
  )(batch_col)


def _layer_body(agg_a, agg_b, xin, sel, W_rel, b_rel, W_root, p_row, h_ref,
                score_ref):
  agg = (agg_a[...] + agg_b[...]) * sel[...]
  h = lax.dot_general(agg, W_rel[...], (((1,), (0,)), ((), ())),
                      preferred_element_type=jnp.float32, precision=lax.Precision.HIGHEST)
  h = h + lax.dot_general(xin[...], W_root[...], (((1,), (0,)), ((), ())),
                          preferred_element_type=jnp.float32, precision=lax.Precision.HIGHEST)
  h = jnp.maximum(h + b_rel[...], 0.0)
  p = p_row[...]
  pn = jnp.sqrt(jnp.sum(p * p)) + 1e-16
  t = lax.dot_general(h, p, (((1,), (1,)), ((), ())),
                      preferred_element_type=jnp.float32, precision=lax.Precision.HIGHEST)
  h_ref[...] = h
  score_ref[...] = jnp.tanh(t / pn)


def _layer(agg_a, agg_b, xin, sel_col, W_rel, b_rel, W_root, p_row,
           interpret=False):
  nblk = NPAD // MB
  blk = lambda i: (i, 0)
  full = lambda i: (0, 0)
  return pl.pallas_call(
      _layer_body,
      grid=(nblk,),
      in_specs=[
          pl.BlockSpec((MB, F), blk),
          pl.BlockSpec((MB, F), blk),
          pl.BlockSpec((MB, F), blk),
          pl.BlockSpec((MB, 1), blk),
          pl.BlockSpec((F, F), full),
          pl.BlockSpec((1, F), full),
          pl.BlockSpec((F, F), full),
          pl.BlockSpec((1, F), full),
      ],
      out_specs=[pl.BlockSpec((MB, F), blk), pl.BlockSpec((MB, 1), blk)],
      out_shape=(jax.ShapeDtypeStruct((NPAD, F), jnp.float32),
                 jax.ShapeDtypeStruct((NPAD, 1), jnp.float32)),
      interpret=interpret,
  )(agg_a, agg_b, xin, sel_col, W_rel, b_rel, W_root, p_row)


def _rank_body(batch_s, start_s, n0_s, batch_col, score_col, prev_col,
               selp_col, score_rows, batch_rows, prev_rows, selp_rows, kf_col,
               h_blk, sel_ref, y_ref):
  bi = pl.program_id(0)
  i0 = bi * RB
  g0 = batch_s[i0]
  gl = batch_s[i0 + RB - 1]
  jlo = start_s[g0]
  jhi = start_s[gl] + n0_s[gl]
  jb0 = jlo >> 8
  jb1 = (jhi + JB - 1) >> 8

  s_i = score_col[...]
  p_i = prev_col[...]
  b_i = batch_col[...]
  sl_i = selp_col[...]
  ii = i0 + lax.broadcasted_iota(jnp.int32, (RB, 1), 0)

  def jbody(jb, rank):
    s_j = score_rows[pl.ds(jb, 1), :]
    b_j = batch_rows[pl.ds(jb, 1), :]
    p_j = prev_rows[pl.ds(jb, 1), :]
    sl_j = selp_rows[pl.ds(jb, 1), :]
    jj = jb * JB + lax.broadcasted_iota(jnp.int32, (1, JB), 1)
    beats = (s_j > s_i) | ((s_j == s_i) &
                           ((p_j > p_i) | ((p_j == p_i) & (jj < ii))))
    cnt = (b_j == b_i) & (sl_j > 0.0) & beats
    return rank + jnp.sum(cnt.astype(jnp.int32), axis=1, keepdims=True)

  rank = lax.fori_loop(jb0, jb1, jbody, jnp.zeros((RB, 1), jnp.int32))
  gg = lax.broadcasted_iota(jnp.int32, (1, 256), 1)
  onehot = (b_i == gg).astype(jnp.float32)
  krow = lax.dot_general(onehot, kf_col[...], (((1,), (0,)), ((), ())),
                         preferred_element_type=jnp.float32, precision=lax.Precision.HIGHEST)
  sel = (sl_i > 0.0) & (rank.astype(jnp.float32) < krow)
  sel_ref[...] = sel.astype(jnp.float32)
  y_ref[...] = jnp.where(sel, s_i * h_blk[...], 0.0)


def _rank(batch_s, start_s, n0_s, batch_col, score_col, prev_col, selp_col,
          score_rows, batch_rows, prev_rows, selp_rows, kf_col, h,
          interpret=False):
  nblk = NPAD // RB
  blk = lambda i, *_: (i, 0)
  full = lambda i, *_: (0, 0)
  nrow = NPAD // JB
  grid_spec = pltpu.PrefetchScalarGridSpec(
      num_scalar_prefetch=3,
      grid=(nblk,),
      in_specs=[
          pl.BlockSpec((RB, 1), blk),
          pl.BlockSpec((RB, 1), blk),
          pl.BlockSpec((RB, 1), blk),
          pl.BlockSpec((RB, 1), blk),
          pl.BlockSpec((nrow, JB), full),
          pl.BlockSpec((nrow, JB), full),
          pl.BlockSpec((nrow, JB), full),
          pl.BlockSpec((nrow, JB), full),
          pl.BlockSpec((256, 1), full),
          pl.BlockSpec((RB, F), blk),
      ],
      out_specs=[pl.BlockSpec((RB, 1), blk), pl.BlockSpec((RB, F), blk)],
  )
  return pl.pallas_call(
      _rank_body,
      grid_spec=grid_spec,
      out_shape=(jax.ShapeDtypeStruct((NPAD, 1), jnp.float32),
                 jax.ShapeDtypeStruct((NPAD, F), jnp.float32)),
      interpret=interpret,
  )(batch_s, start_s, n0_s, batch_col, score_col, prev_col, selp_col,
    score_rows, batch_rows, prev_rows, selp_rows, kf_col, h)


def _pool_body(start_s, n0_s, k1_s, k2_s, y_ref, y2_ref, sel1_ref, sel2_ref,
               out_ref):
  ninf = jnp.float32(-jnp.inf)
  g = pl.program_id(0)
  st = start_s[g]
  n = n0_s[g]
  k1g = k1_s[g]
  k2g = k2_s[g]
  nch = (n + PCH - 1) >> 9

  def cbody(ci, acc):
    mx1, sm1, mx2, sm2 = acc
    off = st + ci * PCH
    yv = y_ref[pl.ds(off, PCH), :]
    y2v = y2_ref[pl.ds(off, PCH), :]
    s1v = sel1_ref[pl.ds(off, PCH), :]
    s2v = sel2_ref[pl.ds(off, PCH), :]
    rowid = ci * PCH + lax.broadcasted_iota(jnp.int32, (PCH, 1), 0)
    valid = rowid < n
    m1 = valid & (s1v > 0.0)
    m2 = valid & (s2v > 0.0)
    mx1 = jnp.maximum(mx1, jnp.max(jnp.where(m1, yv, ninf), axis=0,
                                   keepdims=True))
    sm1 = sm1 + jnp.sum(jnp.where(valid, yv, 0.0), axis=0, keepdims=True)
    mx2 = jnp.maximum(mx2, jnp.max(jnp.where(m2, y2v, ninf), axis=0,
                                   keepdims=True))
    sm2 = sm2 + jnp.sum(jnp.where(valid, y2v, 0.0), axis=0, keepdims=True)
    return mx1, sm1, mx2, sm2

  z = jnp.zeros((1, F), jnp.float32)
  mi = jnp.full((1, F), ninf)
  mx1, sm1, mx2, sm2 = lax.fori_loop(0, nch, cbody, (mi, z, mi, z))
  mx1 = jnp.where(k1g > 0, mx1, 0.0)
  mx2 = jnp.where(k2g > 0, mx2, 0.0)
  mn1 = sm1 / jnp.maximum(k1g, 1).astype(jnp.float32)
  mn2 = sm2 / jnp.maximum(k2g, 1).astype(jnp.float32)
  out_ref[...] = jnp.concatenate([mx1 + mx2, mn1 + mn2], axis=1).reshape(
      1, 1, 2 * F)


def _pool(start_s, n0_s, k1_s, k2_s, y, y2, sel1_col, sel2_col,
          interpret=False):
  full = lambda i, *_: (0, 0)
  grid_spec = pltpu.PrefetchScalarGridSpec(
      num_scalar_prefetch=4,
      grid=(G,),
      in_specs=[
          pl.BlockSpec((NPAD, F), full),
          pl.BlockSpec((NPAD, F), full),
          pl.BlockSpec((NPAD, 1), full),
          pl.BlockSpec((NPAD, 1), full),
      ],
      out_specs=pl.BlockSpec((1, 1, 2 * F), lambda i, *_: (i, 0, 0)),
  )
  return pl.pallas_call(
      _pool_body,
      grid_spec=grid_spec,
      out_shape=jax.ShapeDtypeStruct((G, 1, 2 * F), jnp.float32),
      interpret=interpret,
  )(start_s, n0_s, k1_s, k2_s, y, y2, sel1_col, sel2_col)


def _head_body(zz_ref, Wl1, bl1, Wl2, bl2, out_ref):
  zz = zz_ref[...]
  z1 = lax.dot_general(zz, Wl1[...], (((1,), (0,)), ((), ())),
                       preferred_element_type=jnp.float32, precision=lax.Precision.HIGHEST)
  z1 = jnp.maximum(z1 + bl1[...], 0.0)
  z2 = lax.dot_general(z1, Wl2[...], (((1,), (0,)), ((), ())),
                       preferred_element_type=jnp.float32, precision=lax.Precision.HIGHEST)
  z2 = z2 + bl2[...]
  m = jnp.max(z2, axis=1, keepdims=True)
  e = jnp.exp(z2 - m)
  out_ref[...] = (z2 - m) - jnp.log(jnp.sum(e, axis=1, keepdims=True))


def _head(zz, Wl1, bl1, Wl2, bl2, interpret=False):
  return pl.pallas_call(
      _head_body,
      out_shape=jax.ShapeDtypeStruct((G, C), jnp.float32),
      interpret=interpret,
  )(zz, Wl1, bl1, Wl2, bl2)


def kernel(x, edge_index, batch, W_rel1, b_rel1, W_root1, p1, W_rel2, b_rel2,
           W_root2, p2, W_l1, b_l1, W_l2, b_l2):
  f32 = jnp.float32
  i32 = jnp.int32

  x_pad = jnp.zeros((NPAD, F), f32).at[:N].set(x)
  src2d = jnp.full((EPAD,), N, i32).at[:E].set(edge_index[0]).reshape(
      EPAD // ECH, ECH)
  dst2d = jnp.full((EPAD,), N, i32).at[:E].set(edge_index[1]).reshape(
      EPAD // ECH, ECH)
  batch_pad = jnp.full((NPAD,), PADG, i32).at[:N].set(batch)
  batch_col = batch_pad.reshape(NPAD, 1)
  batch_rows = batch_pad.reshape(NPAD // JB, JB)
  zero_rows = jnp.zeros((ROWS, F), f32)
  ones_col = jnp.ones((NPAD, 1), f32)
  zeros_col = jnp.zeros((NPAD, 1), f32)
  ones_rows = jnp.ones((NPAD // JB, JB), f32)
  zeros_jrows = jnp.zeros((NPAD // JB, JB), f32)

  n0_i, start_i, k1_i, k2_i, k1_f, k2_f = _stats(batch_col)
  n0_s = n0_i.reshape(256)
  start_s = start_i.reshape(256)
  k1_s = k1_i.reshape(256)
  k2_s = k2_i.reshape(256)
  k1_col = k1_f.reshape(256, 1)
  k2_col = k2_f.reshape(256, 1)

  agg1a, agg1b = _sc_scatter_add(x_pad, src2d, dst2d, zero_rows)
  h, score1_col = _layer(agg1a, agg1b, x_pad, ones_col, W_rel1,
                         b_rel1.reshape(1, F), W_root1, p1.reshape(1, F))
  score1_rows = score1_col.reshape(NPAD // JB, JB)
  sel1_col, y = _rank(batch_pad, start_s, n0_s, batch_col, score1_col,
                      zeros_col, ones_col, score1_rows, batch_rows,
                      zeros_jrows, ones_rows, k1_col, h)
  sel1_rows = sel1_col.reshape(NPAD // JB, JB)

  agg2a, agg2b = _sc_scatter_add(y, src2d, dst2d, zero_rows)
  h2, score2_col = _layer(agg2a, agg2b, y, sel1_col, W_rel2,
                          b_rel2.reshape(1, F), W_root2, p2.reshape(1, F))
  score2_rows = score2_col.reshape(NPAD // JB, JB)
  sel2_col, y2 = _rank(batch_pad, start_s, n0_s, batch_col, score2_col,
                       score1_col, sel1_col, score2_rows, batch_rows,
                       score1_rows, sel1_rows, k2_col, h2)

  zz = _pool(start_s, n0_s, k1_s, k2_s, y, y2, sel1_col, sel2_col)
  return _head(zz.reshape(G, 2 * F), W_l1, b_l1.reshape(1, F), W_l2,
               b_l2.reshape(1, C))


# speedup vs baseline: 9.9968x; 1.0996x over previous
import functools

import jax
import jax.numpy as jnp
from jax import lax
from jax.experimental import pallas as pl
from jax.experimental.pallas import tpu as pltpu
from jax.experimental.pallas import tpu_sc as plsc

N = 10000
E = 320000
F = 128
G = 128
C = 10

NPAD = 10752
PADG = 255
EPAD = 327680
NW = 32
EPW = EPAD // NW
ECH = 128
NCH = EPW // ECH
HCH = NCH // 2
ROWS = NPAD // 16

MB = 512
RB = 256
JB = 256
PCH = 512


def _sc_scatter_add(x_pad, src2d, dst2d, zero_rows, interpret=False):
  mesh = plsc.VectorSubcoreMesh(core_axis_name="c", subcore_axis_name="s",
                                num_cores=2, num_subcores=16)

  @functools.partial(
      pl.kernel,
      out_type=(jax.ShapeDtypeStruct((NPAD, F), jnp.float32),
                jax.ShapeDtypeStruct((NPAD, F), jnp.float32)),
      mesh=mesh,
      scratch_types=[
          pltpu.VMEM((HCH, ECH), jnp.int32),
          pltpu.VMEM((HCH, ECH), jnp.int32),
          pltpu.VMEM((ECH, F), jnp.float32),
          pltpu.VMEM((ECH, F), jnp.float32),
          pltpu.VMEM_SHARED((NPAD, F), jnp.float32),
          pltpu.SemaphoreType.DMA,
          pltpu.SemaphoreType.DMA,
      ],
      interpret=interpret,
  )
  def k(x_hbm, src_hbm, dst_hbm, z_hbm, out_a, out_b, src_v, dst_v, rows0,
        rows1, acc_sh, sem0, sem1):
    c = lax.axis_index("c")
    s = lax.axis_index("s")
    wid = s * 2 + c
    pltpu.sync_copy(z_hbm, acc_sh.at[pl.ds(s * ROWS, ROWS)])
    plsc.subcore_barrier()

    for half in range(2):
      base = wid * NCH + half * HCH
      pltpu.sync_copy(src_hbm.at[pl.ds(base, HCH)], src_v)
      pltpu.sync_copy(dst_hbm.at[pl.ds(base, HCH)], dst_v)
      pltpu.async_copy(x_hbm.at[src_v.at[0]], rows0, sem0)

      def body(i, carry):
        pltpu.async_copy(x_hbm.at[src_v.at[i + 1]], rows1, sem1)
        pltpu.make_async_copy(x_hbm.at[src_v.at[i]], rows0, sem0).wait()
        pltpu.sync_copy(rows0, acc_sh.at[dst_v.at[i]], add=True)

        @pl.when(i + 2 < HCH)
        def _():
          pltpu.async_copy(x_hbm.at[src_v.at[i + 2]], rows0, sem0)

        pltpu.make_async_copy(x_hbm.at[src_v.at[i + 1]], rows1, sem1).wait()
        pltpu.sync_copy(rows1, acc_sh.at[dst_v.at[i + 1]], add=True)
        return carry

      lax.fori_loop(0, HCH // 2, lambda j, cc: body(2 * j, cc), 0)

    plsc.subcore_barrier()

    @pl.when(c == 0)
    def _():
      pltpu.sync_copy(acc_sh.at[pl.ds(s * ROWS, ROWS)],
                      out_a.at[pl.ds(s * ROWS, ROWS)])

    @pl.when(c == 1)
    def _():
      pltpu.sync_copy(acc_sh.at[pl.ds(s * ROWS, ROWS)],
                      out_b.at[pl.ds(s * ROWS, ROWS)])

  return k(x_pad, src2d, dst2d, zero_rows)


def _stats_body(batch_ref, n0_ref, start_ref, k1i_ref, k2i_ref, k1f_ref,
                k2f_ref):
  g = lax.broadcasted_iota(jnp.int32, (1, 256), 1)

  def body(i, carry):
    n0, st = carry
    blk = batch_ref[pl.ds(i * 256, 256), :]
    n0 = n0 + jnp.sum((blk == g).astype(jnp.int32), axis=0, keepdims=True)
    st = st + jnp.sum((blk < g).astype(jnp.int32), axis=0, keepdims=True)
    return n0, st

  z = jnp.zeros((1, 256), jnp.int32)
  n0, st = lax.fori_loop(0, NPAD // 256, body, (z, z))
  d1 = (4 * n0 + 4).astype(jnp.float32)
  k1f = jnp.floor(d1 * 0.2 + 0.001)
  k1i = k1f.astype(jnp.int32)
  d2 = (4 * k1i + 4).astype(jnp.float32)
  k2f = jnp.floor(d2 * 0.2 + 0.001)
  k2i = k2f.astype(jnp.int32)
  n0_ref[...] = n0
  start_ref[...] = st
  k1i_ref[...] = k1i
  k2i_ref[...] = k2i
  k1f_ref[...] = k1f
  k2f_ref[...] = k2f


def _stats(batch_col, interpret=False):
  o = jax.ShapeDtypeStruct((1, 256), jnp.int32)
  of = jax.ShapeDtypeStruct((1, 256), jnp.float32)
  return pl.pallas_call(
      _stats_body,
      out_shape=(o, o, o, o, of, of),
      interpret=interpret,
  )(batch_col)


def _layer_body(agg_a, agg_b, xin, sel, W_rel, b_rel, W_root, p_row, h_ref,
                score_ref):
  agg = (agg_a[...] + agg_b[...]) * sel[...]
  h = lax.dot_general(agg, W_rel[...], (((1,), (0,)), ((), ())),
                      preferred_element_type=jnp.float32, precision=lax.Precision.HIGHEST)
  h = h + lax.dot_general(xin[...], W_root[...], (((1,), (0,)), ((), ())),
                          preferred_element_type=jnp.float32, precision=lax.Precision.HIGHEST)
  h = jnp.maximum(h + b_rel[...], 0.0)
  p = p_row[...]
  pn = jnp.sqrt(jnp.sum(p * p)) + 1e-16
  t = lax.dot_general(h, p, (((1,), (1,)), ((), ())),
                      preferred_element_type=jnp.float32, precision=lax.Precision.HIGHEST)
  h_ref[...] = h
  score_ref[...] = jnp.tanh(t / pn)


def _layer(agg_a, agg_b, xin, sel_col, W_rel, b_rel, W_root, p_row,
           interpret=False):
  nblk = NPAD // MB
  blk = lambda i: (i, 0)
  full = lambda i: (0, 0)
  return pl.pallas_call(
      _layer_body,
      grid=(nblk,),
      in_specs=[
          pl.BlockSpec((MB, F), blk),
          pl.BlockSpec((MB, F), blk),
          pl.BlockSpec((MB, F), blk),
          pl.BlockSpec((MB, 1), blk),
          pl.BlockSpec((F, F), full),
          pl.BlockSpec((1, F), full),
          pl.BlockSpec((F, F), full),
          pl.BlockSpec((1, F), full),
      ],
      out_specs=[pl.BlockSpec((MB, F), blk), pl.BlockSpec((MB, 1), blk)],
      out_shape=(jax.ShapeDtypeStruct((NPAD, F), jnp.float32),
                 jax.ShapeDtypeStruct((NPAD, 1), jnp.float32)),
      interpret=interpret,
  )(agg_a, agg_b, xin, sel_col, W_rel, b_rel, W_root, p_row)


def _rank_body(batch_s, start_s, n0_s, batch_col, score_col, prev_col,
               selp_col, score_rows, batch_rows, prev_rows, selp_rows, kf_col,
               h_blk, sel_ref, y_ref):
  bi = pl.program_id(0)
  i0 = bi * RB
  g0 = batch_s[i0]
  gl = batch_s[i0 + RB - 1]
  jlo = start_s[g0]
  jhi = start_s[gl] + n0_s[gl]
  jb0 = jlo >> 8
  jb1 = (jhi + JB - 1) >> 8

  s_i = score_col[...]
  p_i = prev_col[...]
  b_i = batch_col[...]
  sl_i = selp_col[...]
  ii = i0 + lax.broadcasted_iota(jnp.int32, (RB, 1), 0)

  def jbody(jb, rank):
    s_j = score_rows[pl.ds(jb, 1), :]
    b_j = batch_rows[pl.ds(jb, 1), :]
    p_j = prev_rows[pl.ds(jb, 1), :]
    sl_j = selp_rows[pl.ds(jb, 1), :]
    jj = jb * JB + lax.broadcasted_iota(jnp.int32, (1, JB), 1)
    beats = (s_j > s_i) | ((s_j == s_i) &
                           ((p_j > p_i) | ((p_j == p_i) & (jj < ii))))
    cnt = (b_j == b_i) & (sl_j > 0.0) & beats
    return rank + jnp.sum(cnt.astype(jnp.int32), axis=1, keepdims=True)

  rank = lax.fori_loop(jb0, jb1, jbody, jnp.zeros((RB, 1), jnp.int32))
  gg = lax.broadcasted_iota(jnp.int32, (1, 256), 1)
  onehot = (b_i == gg).astype(jnp.float32)
  krow = lax.dot_general(onehot, kf_col[...], (((1,), (0,)), ((), ())),
                         preferred_element_type=jnp.float32, precision=lax.Precision.HIGHEST)
  sel = (sl_i > 0.0) & (rank.astype(jnp.float32) < krow)
  sel_ref[...] = sel.astype(jnp.float32)
  y_ref[...] = jnp.where(sel, s_i * h_blk[...], 0.0)


def _rank(batch_s, start_s, n0_s, batch_col, score_col, prev_col, selp_col,
          score_rows, batch_rows, prev_rows, selp_rows, kf_col, h,
          interpret=False):
  nblk = NPAD // RB
  blk = lambda i, *_: (i, 0)
  full = lambda i, *_: (0, 0)
  nrow = NPAD // JB
  grid_spec = pltpu.PrefetchScalarGridSpec(
      num_scalar_prefetch=3,
      grid=(nblk,),
      in_specs=[
          pl.BlockSpec((RB, 1), blk),
          pl.BlockSpec((RB, 1), blk),
          pl.BlockSpec((RB, 1), blk),
          pl.BlockSpec((RB, 1), blk),
          pl.BlockSpec((nrow, JB), full),
          pl.BlockSpec((nrow, JB), full),
          pl.BlockSpec((nrow, JB), full),
          pl.BlockSpec((nrow, JB), full),
          pl.BlockSpec((256, 1), full),
          pl.BlockSpec((RB, F), blk),
      ],
      out_specs=[pl.BlockSpec((RB, 1), blk), pl.BlockSpec((RB, F), blk)],
  )
  return pl.pallas_call(
      _rank_body,
      grid_spec=grid_spec,
      out_shape=(jax.ShapeDtypeStruct((NPAD, 1), jnp.float32),
                 jax.ShapeDtypeStruct((NPAD, F), jnp.float32)),
      interpret=interpret,
  )(batch_s, start_s, n0_s, batch_col, score_col, prev_col, selp_col,
    score_rows, batch_rows, prev_rows, selp_rows, kf_col, h)


def _pool_body(start_s, n0_s, k1_s, k2_s, y_ref, y2_ref, sel1_ref, sel2_ref,
               out_ref):
  ninf = jnp.float32(-jnp.inf)
  g = pl.program_id(0)
  st = start_s[g]
  n = n0_s[g]
  k1g = k1_s[g]
  k2g = k2_s[g]
  nch = (n + PCH - 1) >> 9

  def cbody(ci, acc):
    mx1, sm1, mx2, sm2 = acc
    off = st + ci * PCH
    yv = y_ref[pl.ds(off, PCH), :]
    y2v = y2_ref[pl.ds(off, PCH), :]
    s1v = sel1_ref[pl.ds(off, PCH), :]
    s2v = sel2_ref[pl.ds(off, PCH), :]
    rowid = ci * PCH + lax.broadcasted_iota(jnp.int32, (PCH, 1), 0)
    valid = rowid < n
    m1 = valid & (s1v > 0.0)
    m2 = valid & (s2v > 0.0)
    mx1 = jnp.maximum(mx1, jnp.max(jnp.where(m1, yv, ninf), axis=0,
                                   keepdims=True))
    sm1 = sm1 + jnp.sum(jnp.where(valid, yv, 0.0), axis=0, keepdims=True)
    mx2 = jnp.maximum(mx2, jnp.max(jnp.where(m2, y2v, ninf), axis=0,
                                   keepdims=True))
    sm2 = sm2 + jnp.sum(jnp.where(valid, y2v, 0.0), axis=0, keepdims=True)
    return mx1, sm1, mx2, sm2

  z = jnp.zeros((1, F), jnp.float32)
  mi = jnp.full((1, F), ninf)
  mx1, sm1, mx2, sm2 = lax.fori_loop(0, nch, cbody, (mi, z, mi, z))
  mx1 = jnp.where(k1g > 0, mx1, 0.0)
  mx2 = jnp.where(k2g > 0, mx2, 0.0)
  mn1 = sm1 / jnp.maximum(k1g, 1).astype(jnp.float32)
  mn2 = sm2 / jnp.maximum(k2g, 1).astype(jnp.float32)
  out_ref[...] = jnp.concatenate([mx1 + mx2, mn1 + mn2], axis=1).reshape(
      1, 1, 2 * F)


def _pool(start_s, n0_s, k1_s, k2_s, y, y2, sel1_col, sel2_col,
          interpret=False):
  full = lambda i, *_: (0, 0)
  grid_spec = pltpu.PrefetchScalarGridSpec(
      num_scalar_prefetch=4,
      grid=(G,),
      in_specs=[
          pl.BlockSpec((NPAD, F), full),
          pl.BlockSpec((NPAD, F), full),
          pl.BlockSpec((NPAD, 1), full),
          pl.BlockSpec((NPAD, 1), full),
      ],
      out_specs=pl.BlockSpec((1, 1, 2 * F), lambda i, *_: (i, 0, 0)),
  )
  return pl.pallas_call(
      _pool_body,
      grid_spec=grid_spec,
      out_shape=jax.ShapeDtypeStruct((G, 1, 2 * F), jnp.float32),
      interpret=interpret,
  )(start_s, n0_s, k1_s, k2_s, y, y2, sel1_col, sel2_col)


def _head_body(zz_ref, Wl1, bl1, Wl2, bl2, out_ref):
  zz = zz_ref[...]
  z1 = lax.dot_general(zz, Wl1[...], (((1,), (0,)), ((), ())),
                       preferred_element_type=jnp.float32, precision=lax.Precision.HIGHEST)
  z1 = jnp.maximum(z1 + bl1[...], 0.0)
  z2 = lax.dot_general(z1, Wl2[...], (((1,), (0,)), ((), ())),
                       preferred_element_type=jnp.float32, precision=lax.Precision.HIGHEST)
  z2 = z2 + bl2[...]
  m = jnp.max(z2, axis=1, keepdims=True)
  e = jnp.exp(z2 - m)
  out_ref[...] = (z2 - m) - jnp.log(jnp.sum(e, axis=1, keepdims=True))


def _head(zz, Wl1, bl1, Wl2, bl2, interpret=False):
  return pl.pallas_call(
      _head_body,
      out_shape=jax.ShapeDtypeStruct((G, C), jnp.float32),
      interpret=interpret,
  )(zz, Wl1, bl1, Wl2, bl2)


def kernel(x, edge_index, batch, W_rel1, b_rel1, W_root1, p1, W_rel2, b_rel2,
           W_root2, p2, W_l1, b_l1, W_l2, b_l2):
  f32 = jnp.float32
  i32 = jnp.int32

  x_pad = jnp.zeros((NPAD, F), f32).at[:N].set(x)
  src2d = jnp.full((EPAD,), N, i32).at[:E].set(edge_index[0]).reshape(
      EPAD // ECH, ECH)
  dst2d = jnp.full((EPAD,), N, i32).at[:E].set(edge_index[1]).reshape(
      EPAD // ECH, ECH)
  batch_pad = jnp.full((NPAD,), PADG, i32).at[:N].set(batch)
  batch_col = batch_pad.reshape(NPAD, 1)
  batch_rows = batch_pad.reshape(NPAD // JB, JB)
  zero_rows = jnp.zeros((ROWS, F), f32)
  ones_col = jnp.ones((NPAD, 1), f32)
  zeros_col = jnp.zeros((NPAD, 1), f32)
  ones_rows = jnp.ones((NPAD // JB, JB), f32)
  zeros_jrows = jnp.zeros((NPAD // JB, JB), f32)

  n0_i, start_i, k1_i, k2_i, k1_f, k2_f = _stats(batch_col)
  n0_s = n0_i.reshape(256)
  start_s = start_i.reshape(256)
  k1_s = k1_i.reshape(256)
  k2_s = k2_i.reshape(256)
  k1_col = k1_f.reshape(256, 1)
  k2_col = k2_f.reshape(256, 1)

  agg1a, agg1b = _sc_scatter_add(x_pad, src2d, dst2d, zero_rows)
  h, score1_col = _layer(agg1a, agg1b, x_pad, ones_col, W_rel1,
                         b_rel1.reshape(1, F), W_root1, p1.reshape(1, F))
  score1_rows = score1_col.reshape(NPAD // JB, JB)
  sel1_col, y = _rank(batch_pad, start_s, n0_s, batch_col, score1_col,
                      zeros_col, ones_col, score1_rows, batch_rows,
                      zeros_jrows, ones_rows, k1_col, h)
  sel1_rows = sel1_col.reshape(NPAD // JB, JB)

  agg2a, agg2b = _sc_scatter_add(y, src2d, dst2d, zero_rows)
  h2, score2_col = _layer(agg2a, agg2b, y, sel1_col, W_rel2,
                          b_rel2.reshape(1, F), W_root2, p2.reshape(1, F))
  score2_rows = score2_col.reshape(NPAD // JB, JB)
  sel2_col, y2 = _rank(batch_pad, start_s, n0_s, batch_col, score2_col,
                       score1_col, sel1_col, score2_rows, batch_rows,
                       score1_rows, sel1_rows, k2_col, h2)

  zz = _pool(start_s, n0_s, k1_s, k2_s, y, y2, sel1_col, sel2_col)
  return _head(zz.reshape(G, 2 * F), W_l1, b_l1.reshape(1, F), W_l2,
               b_l2.reshape(1, C))
